# Initial kernel scaffold; baseline (speedup 1.0000x reference)
import jax
import jax.numpy as jnp
from jax.experimental import pallas as pl


def kernel(x, h, adj_values, W_gates, b_gates, W_cand, b_cand, edge_index):
    raise NotImplementedError("write your pallas kernel here")


# speedup vs baseline: 2.4496x; 2.4496x over previous
import functools

import jax
import jax.numpy as jnp
from jax import lax
from jax.experimental import pallas as pl
from jax.experimental.pallas import tpu as pltpu
from jax.experimental.pallas import tpu_sc as plsc

N = 10000
E = 160000
IN_DIM = 256
HID = 256
CH = 128
NT = 16
NC = 2
EPT = E // NT
BLK = 80
NBLK = EPT // BLK
NP = 10240
ZROWS = 128
ROWS_PT = NP // NT

BN = 1000


def _mm_gates_body(x_ref, h_ref, wx_ref, wh_ref, b_ref, o0, o1, o2, o3):
    acc = jnp.dot(x_ref[...], wx_ref[...], preferred_element_type=jnp.float32)
    acc = acc + jnp.dot(h_ref[...], wh_ref[...],
                        preferred_element_type=jnp.float32)
    acc = acc + b_ref[...]
    o0[...] = acc[:, 0 * CH:1 * CH]
    o1[...] = acc[:, 1 * CH:2 * CH]
    o2[...] = acc[:, 2 * CH:3 * CH]
    o3[...] = acc[:, 3 * CH:4 * CH]


def _mm_gates(x, h, wxT, whT, b):
    grid = (N // BN,)
    return pl.pallas_call(
        _mm_gates_body,
        grid=grid,
        in_specs=[
            pl.BlockSpec((BN, IN_DIM), lambda i: (i, 0)),
            pl.BlockSpec((BN, HID), lambda i: (i, 0)),
            pl.BlockSpec((IN_DIM, 2 * HID), lambda i: (0, 0)),
            pl.BlockSpec((HID, 2 * HID), lambda i: (0, 0)),
            pl.BlockSpec((1, 2 * HID), lambda i: (0, 0)),
        ],
        out_specs=[pl.BlockSpec((BN, CH), lambda i: (i, 0))] * 4,
        out_shape=[jax.ShapeDtypeStruct((N, CH), jnp.float32)] * 4,
    )(x, h, wxT, whT, b)


def _mm_cand_body(x_ref, h_ref, g0, g1, wx_ref, wh_ref, b_ref, o0, o1):
    r0 = jax.nn.sigmoid(g0[...])
    r1 = jax.nn.sigmoid(g1[...])
    rh = jnp.concatenate(
        [r0 * h_ref[:, :CH], r1 * h_ref[:, CH:]], axis=1)
    acc = jnp.dot(x_ref[...], wx_ref[...], preferred_element_type=jnp.float32)
    acc = acc + jnp.dot(rh, wh_ref[...], preferred_element_type=jnp.float32)
    acc = acc + b_ref[...]
    o0[...] = acc[:, :CH]
    o1[...] = acc[:, CH:]


def _mm_cand(x, h, g0, g1, wxT, whT, b):
    grid = (N // BN,)
    return pl.pallas_call(
        _mm_cand_body,
        grid=grid,
        in_specs=[
            pl.BlockSpec((BN, IN_DIM), lambda i: (i, 0)),
            pl.BlockSpec((BN, HID), lambda i: (i, 0)),
            pl.BlockSpec((BN, CH), lambda i: (i, 0)),
            pl.BlockSpec((BN, CH), lambda i: (i, 0)),
            pl.BlockSpec((IN_DIM, HID), lambda i: (0, 0)),
            pl.BlockSpec((HID, HID), lambda i: (0, 0)),
            pl.BlockSpec((1, HID), lambda i: (0, 0)),
        ],
        out_specs=[pl.BlockSpec((BN, CH), lambda i: (i, 0))] * 2,
        out_shape=[jax.ShapeDtypeStruct((N, CH), jnp.float32)] * 2,
    )(x, h, g0, g1, wxT, whT, b)


def _final_body(h_ref, g2, g3, t0, t1, o_ref):
    u = jax.nn.sigmoid(jnp.concatenate([g2[...], g3[...]], axis=1))
    c = jnp.tanh(jnp.concatenate([t0[...], t1[...]], axis=1))
    o_ref[...] = u * h_ref[...] + (1.0 - u) * c


def _final(h, g2, g3, t0, t1):
    grid = (N // BN,)
    return pl.pallas_call(
        _final_body,
        grid=grid,
        in_specs=[
            pl.BlockSpec((BN, HID), lambda i: (i, 0)),
            pl.BlockSpec((BN, CH), lambda i: (i, 0)),
            pl.BlockSpec((BN, CH), lambda i: (i, 0)),
            pl.BlockSpec((BN, CH), lambda i: (i, 0)),
            pl.BlockSpec((BN, CH), lambda i: (i, 0)),
        ],
        out_specs=pl.BlockSpec((BN, HID), lambda i: (i, 0)),
        out_shape=jax.ShapeDtypeStruct((N, HID), jnp.float32),
    )(h, g2, g3, t0, t1)


def _make_spmm(num_chunks):
    cpc = num_chunks // NC
    mesh = plsc.VectorSubcoreMesh(core_axis_name="c", subcore_axis_name="s")

    out_type = [jax.ShapeDtypeStruct((NP, CH), jnp.float32)] * num_chunks
    scratch = [
        pltpu.VMEM((NBLK, BLK), jnp.int32),
        pltpu.VMEM((NBLK, BLK), jnp.int32),
        pltpu.VMEM((BLK * 16,), jnp.float32),
        pltpu.VMEM((BLK, CH), jnp.float32),
        pltpu.VMEM_SHARED((NP, CH), jnp.float32),
        pltpu.SemaphoreType.DMA,
    ]

    @functools.partial(pl.kernel, mesh=mesh, out_type=out_type,
                       scratch_types=scratch)
    def spmm(*refs):
        dense = refs[:num_chunks]
        col_hbm, row_hbm, val_hbm = refs[num_chunks:num_chunks + 3]
        outs = refs[num_chunks + 3:2 * num_chunks + 3]
        col_v, row_v, vrbuf, gbuf, acc, sem = refs[2 * num_chunks + 3:]

        core = lax.axis_index("c")
        sub = lax.axis_index("s")

        pltpu.sync_copy(col_hbm.at[sub], col_v)
        pltpu.sync_copy(row_hbm.at[sub], row_v)

        zero16 = jnp.zeros((16,), jnp.float32)

        def zrow(i, carry):
            for j in range(CH // 16):
                gbuf[i, pl.ds(j * 16, 16)] = zero16
            return carry

        def edge_loop(dref):
            def ebody(b, carry):
                pltpu.sync_copy(
                    val_hbm.at[pl.ds((sub * NBLK + b) * BLK * 16, BLK * 16)],
                    vrbuf)
                cp = pltpu.async_copy(dref.at[col_v.at[b]], gbuf, sem)
                cp.wait()

                def srow(i, c2):
                    v = vrbuf[pl.ds(i * 16, 16)]
                    for j in range(CH // 16):
                        sl = pl.ds(j * 16, 16)
                        gbuf[i, sl] = gbuf[i, sl] * v
                    return c2

                lax.fori_loop(0, BLK, srow, 0)
                pltpu.sync_copy(gbuf, acc.at[row_v.at[b]], add=True)
                return carry

            lax.fori_loop(0, NBLK, ebody, 0)

        for k in range(cpc):
            lax.fori_loop(0, BLK, zrow, 0)
            for z in range(ROWS_PT // BLK):
                pltpu.sync_copy(
                    gbuf, acc.at[pl.ds(sub * ROWS_PT + z * BLK, BLK)])
            plsc.subcore_barrier()

            for cc in range(NC):
                @pl.when(core == cc)
                def _(k=k, cc=cc):
                    edge_loop(dense[cc * cpc + k])

            plsc.subcore_barrier()

            for cc in range(NC):
                @pl.when(core == cc)
                def _(k=k, cc=cc):
                    sl = pl.ds(sub * ROWS_PT, ROWS_PT)
                    pltpu.sync_copy(acc.at[sl], outs[cc * cpc + k].at[sl])

            plsc.subcore_barrier()

    return spmm


_spmm4 = _make_spmm(4)
_spmm2 = _make_spmm(2)


def kernel(x, h, adj_values, W_gates, b_gates, W_cand, b_cand, edge_index):
    row = edge_index[0].astype(jnp.int32).reshape(NT, NBLK, BLK)
    col = edge_index[1].astype(jnp.int32).reshape(NT, NBLK, BLK)
    val = jnp.broadcast_to(
        adj_values.astype(jnp.float32)[:, None], (E, 16)).reshape(E * 16)

    wgxT = W_gates[:, :IN_DIM].T
    wghT = W_gates[:, IN_DIM:].T
    wcxT = W_cand[:, :IN_DIM].T
    wchT = W_cand[:, IN_DIM:].T
    bg = b_gates.reshape(1, 2 * HID)
    bc = b_cand.reshape(1, HID)

    g0, g1, g2, g3 = _mm_gates(x, h, wgxT, wghT, bg)
    s0, s1, s2, s3 = _spmm4(g0, g1, g2, g3, col, row, val)
    c0, c1 = _mm_cand(x, h, s0, s1, wcxT, wchT, bc)
    t0, t1 = _spmm2(c0, c1, col, row, val)
    return _final(h, s2, s3, t0, t1)
